# baseline (device time: 87583 ns/iter reference)
import jax
import jax.numpy as jnp
from jax import lax
from jax.experimental import pallas as pl
from jax.experimental.pallas import tpu as pltpu

N_DEV = 4
N_TILE = 4


def kernel(x, w_mat):
    m_total, k_loc = x.shape
    _, n = w_mat.shape
    mc = m_total // N_DEV
    nh = n // 2
    tw = nh // N_TILE

    def body(
        x_ref,
        w_ref,
        out_ref,
        comm_r_ref,
        comm_l_ref,
        w_bf16_ref,
        stage_ref,
        xb_ref,
        q8s_r_ref,
        q8s_l_ref,
        q8r_r_ref,
        q8r_l_ref,
        sc_send_r_ref,
        sc_send_l_ref,
        sc_recv_r_ref,
        sc_recv_l_ref,
        amax_send_ref,
        amax_recv_ref,
        send_sems_r,
        recv_sems_r,
        send_sems_l,
        recv_sems_l,
        sc_send_sems_r,
        sc_recv_sems_r,
        sc_send_sems_l,
        sc_recv_sems_l,
        stage_sems,
        amax_send_sems,
        amax_recv_sems,
    ):
        r = lax.axis_index("i")
        left = jnp.mod(r - 1, N_DEV)
        right = jnp.mod(r + 1, N_DEV)

        barrier_sem = pltpu.get_barrier_semaphore()
        for nbr in (left, right):
            pl.semaphore_signal(
                barrier_sem,
                inc=1,
                device_id=(nbr,),
                device_id_type=pl.DeviceIdType.MESH,
            )
        pl.semaphore_wait(barrier_sem, 2)

        hk = mc // 2
        cps = {}

        def issue(i, src):
            cp = pltpu.make_async_copy(
                src, stage_ref.at[i % 2], stage_sems.at[i % 2]
            )
            cp.start()
            cps[i] = cp

        def issue_x(i, j, rh):
            issue(i, x_ref.at[pl.ds(j * mc + rh * hk, hk), :])

        def conv(i, dst):
            cps[i].wait()
            dst[...] = stage_ref[i % 2].astype(jnp.bfloat16)

        def dot_b(b, lo, width):
            return jnp.dot(
                xb_ref[b],
                w_bf16_ref[:, lo : lo + width],
                preferred_element_type=jnp.float32,
            )

        def send_tile(comm, ssems, rsems, src_slot, s, t, dev):
            rd = pltpu.make_async_remote_copy(
                src_ref=comm.at[src_slot, :, pl.ds(t * tw, tw)],
                dst_ref=comm.at[s, :, pl.ds(t * tw, tw)],
                send_sem=ssems.at[s, t],
                recv_sem=rsems.at[s, t],
                device_id=(dev,),
                device_id_type=pl.DeviceIdType.MESH,
            )
            rd.start()
            return rd

        jm1 = jnp.mod(r - 1, N_DEV)
        jp1 = jnp.mod(r + 1, N_DEV)
        jp2 = jnp.mod(r + 2, N_DEV)
        issue(0, w_ref.at[0:hk, 0:nh])
        issue(1, w_ref.at[hk : 2 * hk, 0:nh])
        conv(0, w_bf16_ref.at[0:hk, 0:nh])
        issue_x(2, jm1, 0)
        conv(1, w_bf16_ref.at[hk : 2 * hk, 0:nh])
        issue_x(3, jm1, 1)
        conv(2, xb_ref.at[0, 0:hk, :])
        issue(4, w_ref.at[0:hk, nh:n])
        conv(3, xb_ref.at[0, hk:mc, :])
        issue(5, w_ref.at[hk : 2 * hk, nh:n])

        rr = [[None] * N_TILE for _ in range(3)]
        rl = [[None] * N_TILE for _ in range(3)]
        rsc_r = [None] * N_TILE
        rsc_l = [None] * N_TILE

        def send_scale(sc_s, sc_r, ssems, rsems, t, dev):
            rd = pltpu.make_async_remote_copy(
                src_ref=sc_s.at[t],
                dst_ref=sc_r.at[t],
                send_sem=ssems.at[t],
                recv_sem=rsems.at[t],
                device_id=(dev,),
                device_id_type=pl.DeviceIdType.MESH,
            )
            rd.start()
            return rd

        def send_q8(q8s, q8r, ssems, rsems, t, dev):
            rd = pltpu.make_async_remote_copy(
                src_ref=q8s.at[:, pl.ds((t % 2) * tw, tw)],
                dst_ref=q8r.at[:, pl.ds(t * tw, tw)],
                send_sem=ssems.at[0, t],
                recv_sem=rsems.at[0, t],
                device_id=(dev,),
                device_id_type=pl.DeviceIdType.MESH,
            )
            rd.start()
            return rd

        for t in range(N_TILE):
            ts = pl.ds(t * tw, tw)
            pt = dot_b(0, t * tw, tw)
            m_t = jnp.maximum(jnp.max(jnp.abs(pt)), 1e-20)
            sc_send_r_ref[t] = jnp.full((1, 128), m_t / 127.0, jnp.float32)
            rsc_r[t] = send_scale(
                sc_send_r_ref, sc_recv_r_ref, sc_send_sems_r, sc_recv_sems_r,
                t, right,
            )
            if t >= 2:
                rr[0][t - 2].wait_send()
            q8s_r_ref[:, pl.ds((t % 2) * tw, tw)] = jnp.clip(
                jnp.round(pt * (127.0 / m_t)), -127.0, 127.0
            ).astype(jnp.int8)
            rr[0][t] = send_q8(
                q8s_r_ref, q8r_r_ref, send_sems_r, recv_sems_r, t, right
            )
            if t == 0:
                conv(4, w_bf16_ref.at[0:hk, nh:n])
                issue_x(6, jp1, 0)
            elif t == 1:
                conv(5, w_bf16_ref.at[hk : 2 * hk, nh:n])
                issue_x(7, jp1, 1)
            elif t == 2:
                conv(6, xb_ref.at[1, 0:hk, :])
                issue_x(8, jp2, 0)
            else:
                conv(7, xb_ref.at[1, hk:mc, :])
                issue_x(9, jp2, 1)
        for t in range(N_TILE):
            ts = pl.ds(t * tw, tw)
            pt = dot_b(1, nh + t * tw, tw)
            m_t = jnp.maximum(jnp.max(jnp.abs(pt)), 1e-20)
            sc_send_l_ref[t] = jnp.full((1, 128), m_t / 127.0, jnp.float32)
            rsc_l[t] = send_scale(
                sc_send_l_ref, sc_recv_l_ref, sc_send_sems_l, sc_recv_sems_l,
                t, left,
            )
            if t >= 2:
                rl[0][t - 2].wait_send()
            q8s_l_ref[:, pl.ds((t % 2) * tw, tw)] = jnp.clip(
                jnp.round(pt * (127.0 / m_t)), -127.0, 127.0
            ).astype(jnp.int8)
            rl[0][t] = send_q8(
                q8s_l_ref, q8r_l_ref, send_sems_l, recv_sems_l, t, left
            )

        conv(8, xb_ref.at[2, 0:hk, :])
        issue_x(10, r, 0)
        conv(9, xb_ref.at[2, hk:mc, :])
        issue_x(11, r, 1)

        for t in range(N_TILE):
            ts = pl.ds(t * tw, tw)
            rsc_r[t].wait_recv()
            rr[0][t].wait_recv()
            s_in_r = sc_recv_r_ref[t, 0, 0]
            comm_r_ref[0, :, ts] = (
                q8r_r_ref[:, ts].astype(jnp.float32) * s_in_r
                + dot_b(2, t * tw, tw)
            ).astype(jnp.bfloat16)
            rr[1][t] = send_tile(
                comm_r_ref, send_sems_r, recv_sems_r, 0, 1, t, right
            )
            rsc_l[t].wait_recv()
            rl[0][t].wait_recv()
            s_in_l = sc_recv_l_ref[t, 0, 0]
            comm_l_ref[0, :, ts] = (
                q8r_l_ref[:, ts].astype(jnp.float32) * s_in_l
                + dot_b(2, nh + t * tw, tw)
            ).astype(jnp.bfloat16)
            rl[1][t] = send_tile(
                comm_l_ref, send_sems_l, recv_sems_l, 0, 1, t, left
            )

        conv(10, xb_ref.at[2, 0:hk, :])
        conv(11, xb_ref.at[2, hk:mc, :])

        for t in range(N_TILE):
            ts = pl.ds(t * tw, tw)
            rr[1][t].wait_recv()
            comm_r_ref[1, :, ts] = (
                comm_r_ref[1, :, ts].astype(jnp.float32) + dot_b(1, t * tw, tw)
            ).astype(jnp.bfloat16)
            rr[2][t] = send_tile(
                comm_r_ref, send_sems_r, recv_sems_r, 1, 2, t, right
            )
            rl[1][t].wait_recv()
            comm_l_ref[1, :, ts] = (
                comm_l_ref[1, :, ts].astype(jnp.float32)
                + dot_b(0, nh + t * tw, tw)
            ).astype(jnp.bfloat16)
            rl[2][t] = send_tile(
                comm_l_ref, send_sems_l, recv_sems_l, 1, 2, t, left
            )

        maxes = []
        for t in range(N_TILE):
            ts = pl.ds(t * tw, tw)
            cs = slice(t * tw, (t + 1) * tw)
            rr[2][t].wait_recv()
            ya_t = jnp.maximum(
                comm_r_ref[2, :, ts].astype(jnp.float32) + dot_b(2, t * tw, tw),
                0.0,
            )
            maxes.append(jnp.max(ya_t))
            out_ref[:, ts] = ya_t.astype(jnp.bfloat16)
            rl[2][t].wait_recv()
            yb_t = jnp.maximum(
                comm_l_ref[2, :, ts].astype(jnp.float32)
                + dot_b(2, nh + t * tw, tw),
                0.0,
            )
            maxes.append(jnp.max(yb_t))
            out_ref[:, pl.ds(nh + t * tw, tw)] = yb_t.astype(jnp.bfloat16)
        m_loc = maxes[0]
        for m in maxes[1:]:
            m_loc = jnp.maximum(m_loc, m)
        amax_send_ref[...] = jnp.full((8, 128), m_loc, jnp.float32)
        rdmas = []
        for off in (1, 2, 3):
            tgt = jnp.mod(r + off, N_DEV)
            a_rdma = pltpu.make_async_remote_copy(
                src_ref=amax_send_ref,
                dst_ref=amax_recv_ref.at[off],
                send_sem=amax_send_sems.at[off - 1],
                recv_sem=amax_recv_sems.at[off - 1],
                device_id=(tgt,),
                device_id_type=pl.DeviceIdType.MESH,
            )
            a_rdma.start()
            rdmas.append(a_rdma)
        for s in range(3):
            for t in range(N_TILE):
                if s == 0 and t < N_TILE - 2:
                    continue
                rr[s][t].wait_send()
                rl[s][t].wait_send()
        for t in range(N_TILE):
            rsc_r[t].wait_send()
            rsc_l[t].wait_send()
        for a_rdma in rdmas:
            a_rdma.wait_send()
            a_rdma.wait_recv()

        amax_g = jnp.maximum(
            jnp.maximum(m_loc, amax_recv_ref[1, 0, 0]),
            jnp.maximum(amax_recv_ref[2, 0, 0], amax_recv_ref[3, 0, 0]),
        )

        scale = amax_g / 127.0
        inv = 127.0 / amax_g
        for q in range(4):
            qs = pl.ds(q * (n // 4), n // 4)
            out_ref[:, qs] = (
                jnp.clip(
                    jnp.round(out_ref[:, qs].astype(jnp.float32) * inv),
                    -127.0,
                    127.0,
                )
                * scale
            ).astype(jnp.bfloat16)

    return pl.pallas_call(
        body,
        out_shape=jax.ShapeDtypeStruct((mc, n), jnp.bfloat16),
        in_specs=[
            pl.BlockSpec(memory_space=pl.ANY),
            pl.BlockSpec(memory_space=pl.ANY),
        ],
        out_specs=pl.BlockSpec(memory_space=pltpu.VMEM),
        scratch_shapes=[
            pltpu.VMEM((3, mc, nh), jnp.bfloat16),
            pltpu.VMEM((3, mc, nh), jnp.bfloat16),
            pltpu.VMEM((k_loc, n), jnp.bfloat16),
            pltpu.VMEM((2, mc // 2, k_loc), jnp.float32),
            pltpu.VMEM((3, mc, k_loc), jnp.bfloat16),
            pltpu.VMEM((mc, 2 * nh // N_TILE), jnp.int8),
            pltpu.VMEM((mc, 2 * nh // N_TILE), jnp.int8),
            pltpu.VMEM((mc, nh), jnp.int8),
            pltpu.VMEM((mc, nh), jnp.int8),
            pltpu.VMEM((N_TILE, 1, 128), jnp.float32),
            pltpu.VMEM((N_TILE, 1, 128), jnp.float32),
            pltpu.VMEM((N_TILE, 1, 128), jnp.float32),
            pltpu.VMEM((N_TILE, 1, 128), jnp.float32),
            pltpu.VMEM((8, 128), jnp.float32),
            pltpu.VMEM((N_DEV, 8, 128), jnp.float32),
            pltpu.SemaphoreType.DMA((3, N_TILE)),
            pltpu.SemaphoreType.DMA((3, N_TILE)),
            pltpu.SemaphoreType.DMA((3, N_TILE)),
            pltpu.SemaphoreType.DMA((3, N_TILE)),
            pltpu.SemaphoreType.DMA((N_TILE,)),
            pltpu.SemaphoreType.DMA((N_TILE,)),
            pltpu.SemaphoreType.DMA((N_TILE,)),
            pltpu.SemaphoreType.DMA((N_TILE,)),
            pltpu.SemaphoreType.DMA((2,)),
            pltpu.SemaphoreType.DMA((N_DEV - 1,)),
            pltpu.SemaphoreType.DMA((N_DEV - 1,)),
        ],
        compiler_params=pltpu.CompilerParams(collective_id=0),
    )(x, w_mat)


# device time: 85643 ns/iter; 1.0227x vs baseline; 1.0227x over previous
import jax
import jax.numpy as jnp
from jax import lax
from jax.experimental import pallas as pl
from jax.experimental.pallas import tpu as pltpu

N_DEV = 4
N_TILE = 2


def kernel(x, w_mat):
    m_total, k_loc = x.shape
    _, n = w_mat.shape
    mc = m_total // N_DEV
    nh = n // 2
    tw = nh // N_TILE

    def body(
        x_ref,
        w_ref,
        out_ref,
        comm_r_ref,
        comm_l_ref,
        w_bf16_ref,
        stage_ref,
        xb_ref,
        q8s_r_ref,
        q8s_l_ref,
        q8r_r_ref,
        q8r_l_ref,
        sc_send_r_ref,
        sc_send_l_ref,
        sc_recv_r_ref,
        sc_recv_l_ref,
        amax_send_ref,
        amax_recv_ref,
        send_sems_r,
        recv_sems_r,
        send_sems_l,
        recv_sems_l,
        sc_send_sems_r,
        sc_recv_sems_r,
        sc_send_sems_l,
        sc_recv_sems_l,
        stage_sems,
        amax_send_sems,
        amax_recv_sems,
    ):
        r = lax.axis_index("i")
        left = jnp.mod(r - 1, N_DEV)
        right = jnp.mod(r + 1, N_DEV)

        barrier_sem = pltpu.get_barrier_semaphore()
        for nbr in (left, right):
            pl.semaphore_signal(
                barrier_sem,
                inc=1,
                device_id=(nbr,),
                device_id_type=pl.DeviceIdType.MESH,
            )
        pl.semaphore_wait(barrier_sem, 2)

        hk = mc // 2
        cps = {}

        def issue(i, src):
            cp = pltpu.make_async_copy(
                src, stage_ref.at[i % 2], stage_sems.at[i % 2]
            )
            cp.start()
            cps[i] = cp

        def issue_x(i, j, rh):
            issue(i, x_ref.at[pl.ds(j * mc + rh * hk, hk), :])

        def conv(i, dst):
            cps[i].wait()
            dst[...] = stage_ref[i % 2].astype(jnp.bfloat16)

        def dot_b(b, lo, width):
            return jnp.dot(
                xb_ref[b],
                w_bf16_ref[:, lo : lo + width],
                preferred_element_type=jnp.float32,
            )

        def send_tile(comm, ssems, rsems, src_slot, s, t, dev):
            rd = pltpu.make_async_remote_copy(
                src_ref=comm.at[src_slot, :, pl.ds(t * tw, tw)],
                dst_ref=comm.at[s, :, pl.ds(t * tw, tw)],
                send_sem=ssems.at[s, t],
                recv_sem=rsems.at[s, t],
                device_id=(dev,),
                device_id_type=pl.DeviceIdType.MESH,
            )
            rd.start()
            return rd

        jm1 = jnp.mod(r - 1, N_DEV)
        jp1 = jnp.mod(r + 1, N_DEV)
        jp2 = jnp.mod(r + 2, N_DEV)
        issue(0, w_ref.at[0:hk, 0:nh])
        issue(1, w_ref.at[hk : 2 * hk, 0:nh])
        conv(0, w_bf16_ref.at[0:hk, 0:nh])
        issue_x(2, jm1, 0)
        conv(1, w_bf16_ref.at[hk : 2 * hk, 0:nh])
        issue_x(3, jm1, 1)
        conv(2, xb_ref.at[0, 0:hk, :])
        issue(4, w_ref.at[0:hk, nh:n])
        conv(3, xb_ref.at[0, hk:mc, :])
        issue(5, w_ref.at[hk : 2 * hk, nh:n])

        rr = [[None] * N_TILE for _ in range(3)]
        rl = [[None] * N_TILE for _ in range(3)]
        rsc_r = [None] * N_TILE
        rsc_l = [None] * N_TILE

        def send_scale(sc_s, sc_r, ssems, rsems, t, dev):
            rd = pltpu.make_async_remote_copy(
                src_ref=sc_s.at[t],
                dst_ref=sc_r.at[t],
                send_sem=ssems.at[t],
                recv_sem=rsems.at[t],
                device_id=(dev,),
                device_id_type=pl.DeviceIdType.MESH,
            )
            rd.start()
            return rd

        def send_q8(q8s, q8r, ssems, rsems, t, dev):
            rd = pltpu.make_async_remote_copy(
                src_ref=q8s.at[:, pl.ds(t * tw, tw)],
                dst_ref=q8r.at[:, pl.ds(t * tw, tw)],
                send_sem=ssems.at[0, t],
                recv_sem=rsems.at[0, t],
                device_id=(dev,),
                device_id_type=pl.DeviceIdType.MESH,
            )
            rd.start()
            return rd

        fetch_steps = [
            lambda: (conv(4, w_bf16_ref.at[0:hk, nh:n]), issue_x(6, jp1, 0)),
            lambda: (
                conv(5, w_bf16_ref.at[hk : 2 * hk, nh:n]),
                issue_x(7, jp1, 1),
            ),
            lambda: (conv(6, xb_ref.at[1, 0:hk, :]), issue_x(8, jp2, 0)),
            lambda: (conv(7, xb_ref.at[1, hk:mc, :]), issue_x(9, jp2, 1)),
        ]

        for t in range(N_TILE):
            ts = pl.ds(t * tw, tw)
            pt = dot_b(0, t * tw, tw)
            m_t = jnp.maximum(jnp.max(jnp.abs(pt)), 1e-20)
            sc_send_r_ref[t] = jnp.full((1, 128), m_t / 127.0, jnp.float32)
            rsc_r[t] = send_scale(
                sc_send_r_ref, sc_recv_r_ref, sc_send_sems_r, sc_recv_sems_r,
                t, right,
            )
            q8s_r_ref[:, ts] = jnp.clip(
                jnp.round(pt * (127.0 / m_t)), -127.0, 127.0
            ).astype(jnp.int8)
            rr[0][t] = send_q8(
                q8s_r_ref, q8r_r_ref, send_sems_r, recv_sems_r, t, right
            )
            if t < len(fetch_steps):
                fetch_steps[t]()
        for step in fetch_steps[N_TILE:]:
            step()
        for t in range(N_TILE):
            ts = pl.ds(t * tw, tw)
            pt = dot_b(1, nh + t * tw, tw)
            m_t = jnp.maximum(jnp.max(jnp.abs(pt)), 1e-20)
            sc_send_l_ref[t] = jnp.full((1, 128), m_t / 127.0, jnp.float32)
            rsc_l[t] = send_scale(
                sc_send_l_ref, sc_recv_l_ref, sc_send_sems_l, sc_recv_sems_l,
                t, left,
            )
            q8s_l_ref[:, ts] = jnp.clip(
                jnp.round(pt * (127.0 / m_t)), -127.0, 127.0
            ).astype(jnp.int8)
            rl[0][t] = send_q8(
                q8s_l_ref, q8r_l_ref, send_sems_l, recv_sems_l, t, left
            )

        conv(8, xb_ref.at[0, 0:hk, :])
        issue_x(10, jm1, 0)
        conv(9, xb_ref.at[0, hk:mc, :])
        issue_x(11, jm1, 1)

        for t in range(N_TILE):
            ts = pl.ds(t * tw, tw)
            rsc_r[t].wait_recv()
            rr[0][t].wait_recv()
            s_in_r = sc_recv_r_ref[t, 0, 0]
            comm_r_ref[0, :, ts] = (
                q8r_r_ref[:, ts].astype(jnp.float32) * s_in_r
                + dot_b(0, t * tw, tw)
            ).astype(jnp.bfloat16)
            rr[1][t] = send_tile(
                comm_r_ref, send_sems_r, recv_sems_r, 0, 1, t, right
            )
            rsc_l[t].wait_recv()
            rl[0][t].wait_recv()
            s_in_l = sc_recv_l_ref[t, 0, 0]
            comm_l_ref[0, :, ts] = (
                q8r_l_ref[:, ts].astype(jnp.float32) * s_in_l
                + dot_b(0, nh + t * tw, tw)
            ).astype(jnp.bfloat16)
            rl[1][t] = send_tile(
                comm_l_ref, send_sems_l, recv_sems_l, 0, 1, t, left
            )

        conv(10, xb_ref.at[0, 0:hk, :])
        issue_x(12, r, 0)
        conv(11, xb_ref.at[0, hk:mc, :])
        issue_x(13, r, 1)

        for t in range(N_TILE):
            ts = pl.ds(t * tw, tw)
            rr[1][t].wait_recv()
            comm_r_ref[1, :, ts] = (
                comm_r_ref[1, :, ts].astype(jnp.float32) + dot_b(1, t * tw, tw)
            ).astype(jnp.bfloat16)
            rr[2][t] = send_tile(
                comm_r_ref, send_sems_r, recv_sems_r, 1, 2, t, right
            )
            rl[1][t].wait_recv()
            comm_l_ref[1, :, ts] = (
                comm_l_ref[1, :, ts].astype(jnp.float32)
                + dot_b(0, nh + t * tw, tw)
            ).astype(jnp.bfloat16)
            rl[2][t] = send_tile(
                comm_l_ref, send_sems_l, recv_sems_l, 1, 2, t, left
            )

        conv(12, xb_ref.at[1, 0:hk, :])
        conv(13, xb_ref.at[1, hk:mc, :])

        maxes = []
        for t in range(N_TILE):
            ts = pl.ds(t * tw, tw)
            cs = slice(t * tw, (t + 1) * tw)
            rr[2][t].wait_recv()
            ya_t = jnp.maximum(
                comm_r_ref[2, :, ts].astype(jnp.float32) + dot_b(1, t * tw, tw),
                0.0,
            )
            maxes.append(jnp.max(ya_t))
            out_ref[:, ts] = ya_t.astype(jnp.bfloat16)
            rl[2][t].wait_recv()
            yb_t = jnp.maximum(
                comm_l_ref[2, :, ts].astype(jnp.float32)
                + dot_b(1, nh + t * tw, tw),
                0.0,
            )
            maxes.append(jnp.max(yb_t))
            out_ref[:, pl.ds(nh + t * tw, tw)] = yb_t.astype(jnp.bfloat16)
        m_loc = maxes[0]
        for m in maxes[1:]:
            m_loc = jnp.maximum(m_loc, m)
        amax_send_ref[...] = jnp.full((8, 128), m_loc, jnp.float32)
        rdmas = []
        for off in (1, 2, 3):
            tgt = jnp.mod(r + off, N_DEV)
            a_rdma = pltpu.make_async_remote_copy(
                src_ref=amax_send_ref,
                dst_ref=amax_recv_ref.at[off],
                send_sem=amax_send_sems.at[off - 1],
                recv_sem=amax_recv_sems.at[off - 1],
                device_id=(tgt,),
                device_id_type=pl.DeviceIdType.MESH,
            )
            a_rdma.start()
            rdmas.append(a_rdma)
        for s in range(3):
            for t in range(N_TILE):
                rr[s][t].wait_send()
                rl[s][t].wait_send()
        for t in range(N_TILE):
            rsc_r[t].wait_send()
            rsc_l[t].wait_send()
        for a_rdma in rdmas:
            a_rdma.wait_send()
            a_rdma.wait_recv()

        amax_g = jnp.maximum(
            jnp.maximum(m_loc, amax_recv_ref[1, 0, 0]),
            jnp.maximum(amax_recv_ref[2, 0, 0], amax_recv_ref[3, 0, 0]),
        )

        scale = amax_g / 127.0
        inv = 127.0 / amax_g
        for q in range(4):
            qs = pl.ds(q * (n // 4), n // 4)
            out_ref[:, qs] = (
                jnp.clip(
                    jnp.round(out_ref[:, qs].astype(jnp.float32) * inv),
                    -127.0,
                    127.0,
                )
                * scale
            ).astype(jnp.bfloat16)

    return pl.pallas_call(
        body,
        out_shape=jax.ShapeDtypeStruct((mc, n), jnp.bfloat16),
        in_specs=[
            pl.BlockSpec(memory_space=pl.ANY),
            pl.BlockSpec(memory_space=pl.ANY),
        ],
        out_specs=pl.BlockSpec(memory_space=pltpu.VMEM),
        scratch_shapes=[
            pltpu.VMEM((3, mc, nh), jnp.bfloat16),
            pltpu.VMEM((3, mc, nh), jnp.bfloat16),
            pltpu.VMEM((k_loc, n), jnp.bfloat16),
            pltpu.VMEM((2, mc // 2, k_loc), jnp.float32),
            pltpu.VMEM((2, mc, k_loc), jnp.bfloat16),
            pltpu.VMEM((mc, nh), jnp.int8),
            pltpu.VMEM((mc, nh), jnp.int8),
            pltpu.VMEM((mc, nh), jnp.int8),
            pltpu.VMEM((mc, nh), jnp.int8),
            pltpu.VMEM((N_TILE, 1, 128), jnp.float32),
            pltpu.VMEM((N_TILE, 1, 128), jnp.float32),
            pltpu.VMEM((N_TILE, 1, 128), jnp.float32),
            pltpu.VMEM((N_TILE, 1, 128), jnp.float32),
            pltpu.VMEM((8, 128), jnp.float32),
            pltpu.VMEM((N_DEV, 8, 128), jnp.float32),
            pltpu.SemaphoreType.DMA((3, N_TILE)),
            pltpu.SemaphoreType.DMA((3, N_TILE)),
            pltpu.SemaphoreType.DMA((3, N_TILE)),
            pltpu.SemaphoreType.DMA((3, N_TILE)),
            pltpu.SemaphoreType.DMA((N_TILE,)),
            pltpu.SemaphoreType.DMA((N_TILE,)),
            pltpu.SemaphoreType.DMA((N_TILE,)),
            pltpu.SemaphoreType.DMA((N_TILE,)),
            pltpu.SemaphoreType.DMA((2,)),
            pltpu.SemaphoreType.DMA((N_DEV - 1,)),
            pltpu.SemaphoreType.DMA((N_DEV - 1,)),
        ],
        compiler_params=pltpu.CompilerParams(collective_id=0),
    )(x, w_mat)
